# baseline (speedup 1.0000x reference)
import functools

import jax
import jax.numpy as jnp
import numpy as np
from jax import lax
from jax.experimental import pallas as pl
from jax.experimental.pallas import tpu as pltpu
from jax.experimental.pallas import tpu_sc as plsc

_POOLING_COUNTS = (15, 8, 8)
_N = 64
_B = 32
_D = 512
_NC = 2
_NS = 16
_LANES = 16


def _mask2d_np(N, pooling_counts):
    m = np.zeros((N, N), dtype=bool)
    m[np.arange(N), np.arange(N)] = True
    stride, offset = 1, 0
    for c in pooling_counts:
        for _ in range(c):
            offset += stride
            i = np.arange(0, N - offset, stride)
            m[i, i + offset] = True
        stride *= 2
    return m


def _schedule(N, pooling_counts):
    poolers = [(2, 1) for _ in range(pooling_counts[0])]
    for c in pooling_counts[1:]:
        poolers.append((3, 2))
        poolers.extend([(2, 1) for _ in range(c - 1)])
    offs = []
    stride, offset = 1, 0
    for c in pooling_counts:
        for _ in range(c):
            offset += stride
            offs.append((offset, stride))
        stride *= 2
    sched = []
    L = N
    for (k, s), (off, st) in zip(poolers, offs):
        L = (L - k) // s + 1
        sched.append((k, s, off, st, L))
    return sched


_MASK = _mask2d_np(_N, _POOLING_COUNTS)
_SCHED = _schedule(_N, _POOLING_COUNTS)


def _quarter_masks():
    qs = [set() for _ in range(4)]
    for i, j in zip(*np.where(_MASK)):
        qs[int(i) // 16].add((int(i) % 16, int(j) // 8, int(j) % 8))
    return qs


_QMASK = _quarter_masks()
_REZERO = {
    0: sorted(_QMASK[2] - _QMASK[0]),
    2: sorted(_QMASK[0] - _QMASK[2]),
    1: sorted(_QMASK[3] - _QMASK[1]),
    3: sorted(_QMASK[1] - _QMASK[3]),
}


def _sc_body(xt_hbm, zeros_hbm, out_hbm, xv, obuf_a, obuf_b, sem_a, sem_b):
    c = lax.axis_index("c")
    s = lax.axis_index("s")
    b = s * _NC + c
    pltpu.make_async_copy(zeros_hbm, obuf_a, sem_a).start()
    pltpu.make_async_copy(zeros_hbm, obuf_b, sem_b).start()
    xbase = lax.iota(jnp.int32, _LANES) * _N
    zero16 = jnp.zeros((_LANES,), dtype=jnp.float32)

    def unit(p, carry):
        pltpu.sync_copy(xt_hbm.at[b, pl.ds(p * 2 * _LANES * _N, 2 * _LANES * _N)], xv)
        vals = [[] for _ in range(4)]

        def put(v0, v1, i, j):
            vals[i // 16].append((i % 16, j // 8, j % 8, v0, v1))

        rowp = []
        for c2 in range(2):
            rowp.append(
                [plsc.load_gather(xv, [xbase + (c2 * 1024 + n)]) for n in range(_N)]
            )
        for i in range(_N):
            put(rowp[0][i], rowp[1][i], i, i)
        cur0, cur1 = rowp
        for k, sp, off, st, L in _SCHED:
            new0, new1 = [], []
            for t in range(L):
                v0 = jnp.maximum(cur0[sp * t], cur0[sp * t + 1])
                v1 = jnp.maximum(cur1[sp * t], cur1[sp * t + 1])
                if k == 3:
                    v0 = jnp.maximum(v0, cur0[sp * t + 2])
                    v1 = jnp.maximum(v1, cur1[sp * t + 2])
                new0.append(v0)
                new1.append(v1)
                put(v0, v1, st * t, st * t + off)
            cur0, cur1 = new0, new1
        dt = p // 4
        dl0 = (p % 4) * 2 * _LANES
        for q in range(4):
            buf, sem = (obuf_a, sem_a) if q % 2 == 0 else (obuf_b, sem_b)
            pltpu.make_async_copy(zeros_hbm, buf, sem).wait()
            for (ii, jt, jl) in _REZERO[q]:
                buf[ii, jt, jl, 0:_LANES] = zero16
                buf[ii, jt, jl, _LANES : 2 * _LANES] = zero16
            for (ii, jt, jl, v0, v1) in vals[q]:
                buf[ii, jt, jl, 0:_LANES] = v0
                buf[ii, jt, jl, _LANES : 2 * _LANES] = v1
            dst = out_hbm.at[b, pl.ds(q * 16, 16), :, dt, :, pl.ds(dl0, 2 * _LANES)]
            pltpu.make_async_copy(buf, dst, sem).start()
        return carry

    lax.fori_loop(0, _D // (2 * _LANES), unit, 0)
    pltpu.make_async_copy(zeros_hbm, obuf_a, sem_a).wait()
    pltpu.make_async_copy(zeros_hbm, obuf_b, sem_b).wait()


@functools.partial(jax.jit, static_argnames=())
def kernel(x):
    B, D, N = x.shape
    xt = x.reshape(B, D * N)
    zeros = jnp.zeros((16, N // 8, 8, 2 * _LANES), dtype=x.dtype)
    mesh = plsc.VectorSubcoreMesh(
        core_axis_name="c", subcore_axis_name="s", num_cores=_NC, num_subcores=_NS
    )
    fn = pl.kernel(
        _sc_body,
        out_type=jax.ShapeDtypeStruct((B, N, N // 8, D // 128, 8, 128), x.dtype),
        mesh=mesh,
        scratch_types=[
            pltpu.VMEM((2 * _LANES * N,), x.dtype),
            pltpu.VMEM((16, N // 8, 8, 2 * _LANES), x.dtype),
            pltpu.VMEM((16, N // 8, 8, 2 * _LANES), x.dtype),
            pltpu.SemaphoreType.DMA,
            pltpu.SemaphoreType.DMA,
        ],
        compiler_params=pltpu.CompilerParams(
            needs_layout_passes=False, use_tc_tiling_on_sc=False
        ),
    )
    out6 = fn(xt, zeros)
    map2d = out6.transpose(0, 3, 5, 1, 2, 4).reshape(B, D, N, N)
    mask2d = jnp.broadcast_to(jnp.asarray(_MASK)[None, None, :, :], (B, 1, N, N))
    return (map2d, mask2d)


# speedup vs baseline: 1.2850x; 1.2850x over previous
import functools

import jax
import jax.numpy as jnp
import numpy as np
from jax import lax
from jax.experimental import pallas as pl
from jax.experimental.pallas import tpu as pltpu
from jax.experimental.pallas import tpu_sc as plsc

_POOLING_COUNTS = (15, 8, 8)
_N = 64
_B = 32
_D = 512
_NC = 2
_NS = 16
_LANES = 16


def _mask2d_np(N, pooling_counts):
    m = np.zeros((N, N), dtype=bool)
    m[np.arange(N), np.arange(N)] = True
    stride, offset = 1, 0
    for c in pooling_counts:
        for _ in range(c):
            offset += stride
            i = np.arange(0, N - offset, stride)
            m[i, i + offset] = True
        stride *= 2
    return m


def _schedule(N, pooling_counts):
    poolers = [(2, 1) for _ in range(pooling_counts[0])]
    for c in pooling_counts[1:]:
        poolers.append((3, 2))
        poolers.extend([(2, 1) for _ in range(c - 1)])
    offs = []
    stride, offset = 1, 0
    for c in pooling_counts:
        for _ in range(c):
            offset += stride
            offs.append((offset, stride))
        stride *= 2
    sched = []
    L = N
    for (k, s), (off, st) in zip(poolers, offs):
        L = (L - k) // s + 1
        sched.append((k, s, off, st, L))
    return sched


_MASK = _mask2d_np(_N, _POOLING_COUNTS)
_SCHED = _schedule(_N, _POOLING_COUNTS)


def _sc_body(xt_hbm, zeros_hbm, out_hbm, xv, obuf_a, obuf_b, sem_a, sem_b):
    c = lax.axis_index("c")
    s = lax.axis_index("s")
    b = s * _NC + c
    pltpu.make_async_copy(zeros_hbm, obuf_a, sem_a).start()
    pltpu.make_async_copy(zeros_hbm, obuf_b, sem_b).start()
    xbase = lax.iota(jnp.int32, _LANES) * _N

    def unit(dc, carry):
        pltpu.sync_copy(xt_hbm.at[b, pl.ds(dc * _LANES * _N, _LANES * _N)], xv)
        rows = [plsc.load_gather(xv, [xbase + n]) for n in range(_N)]
        pltpu.make_async_copy(zeros_hbm, obuf_a, sem_a).wait()
        pltpu.make_async_copy(zeros_hbm, obuf_b, sem_b).wait()

        def put(v, i, j):
            if i < _N // 2:
                obuf_a[i, j // 8, j % 8, :] = v
            else:
                obuf_b[i - _N // 2, j // 8, j % 8, :] = v

        for i in range(_N):
            put(rows[i], i, i)
        cur = rows
        for k, sp, off, st, L in _SCHED:
            new = []
            for t in range(L):
                v = jnp.maximum(cur[sp * t], cur[sp * t + 1])
                if k == 3:
                    v = jnp.maximum(v, cur[sp * t + 2])
                new.append(v)
                put(v, st * t, st * t + off)
            cur = new
        dt = dc // 8
        dl0 = (dc % 8) * _LANES
        half = _N // 2
        dst_a = out_hbm.at[b, pl.ds(0, half), :, dt, :, pl.ds(dl0, _LANES)]
        dst_b = out_hbm.at[b, pl.ds(half, half), :, dt, :, pl.ds(dl0, _LANES)]
        pltpu.make_async_copy(obuf_a, dst_a, sem_a).start()
        pltpu.make_async_copy(obuf_b, dst_b, sem_b).start()
        return carry

    lax.fori_loop(0, _D // _LANES, unit, 0)
    pltpu.make_async_copy(zeros_hbm, obuf_a, sem_a).wait()
    pltpu.make_async_copy(zeros_hbm, obuf_b, sem_b).wait()


@functools.partial(jax.jit, static_argnames=())
def kernel(x):
    B, D, N = x.shape
    xt = x.reshape(B, D * N)
    zeros = jnp.zeros((N // 2, N // 8, 8, _LANES), dtype=x.dtype)
    mesh = plsc.VectorSubcoreMesh(
        core_axis_name="c", subcore_axis_name="s", num_cores=_NC, num_subcores=_NS
    )
    fn = pl.kernel(
        _sc_body,
        out_type=jax.ShapeDtypeStruct((B, N, N // 8, D // 128, 8, 128), x.dtype),
        mesh=mesh,
        scratch_types=[
            pltpu.VMEM((_LANES * N,), x.dtype),
            pltpu.VMEM((N // 2, N // 8, 8, _LANES), x.dtype),
            pltpu.VMEM((N // 2, N // 8, 8, _LANES), x.dtype),
            pltpu.SemaphoreType.DMA,
            pltpu.SemaphoreType.DMA,
        ],
        compiler_params=pltpu.CompilerParams(
            needs_layout_passes=False, use_tc_tiling_on_sc=False
        ),
    )
    out6 = fn(xt, zeros)
    map2d = out6.transpose(0, 3, 5, 1, 2, 4).reshape(B, D, N, N)
    mask2d = jnp.broadcast_to(jnp.asarray(_MASK)[None, None, :, :], (B, 1, N, N))
    return (map2d, mask2d)


# speedup vs baseline: 1.4259x; 1.1096x over previous
import functools

import jax
import jax.numpy as jnp
import numpy as np
from jax import lax
from jax.experimental import pallas as pl
from jax.experimental.pallas import tpu as pltpu
from jax.experimental.pallas import tpu_sc as plsc

_POOLING_COUNTS = (15, 8, 8)
_N = 64
_B = 32
_D = 512
_NC = 2
_NS = 16
_LANES = 16


def _mask2d_np(N, pooling_counts):
    m = np.zeros((N, N), dtype=bool)
    m[np.arange(N), np.arange(N)] = True
    stride, offset = 1, 0
    for c in pooling_counts:
        for _ in range(c):
            offset += stride
            i = np.arange(0, N - offset, stride)
            m[i, i + offset] = True
        stride *= 2
    return m


def _schedule(N, pooling_counts):
    poolers = [(2, 1) for _ in range(pooling_counts[0])]
    for c in pooling_counts[1:]:
        poolers.append((3, 2))
        poolers.extend([(2, 1) for _ in range(c - 1)])
    offs = []
    stride, offset = 1, 0
    for c in pooling_counts:
        for _ in range(c):
            offset += stride
            offs.append((offset, stride))
        stride *= 2
    sched = []
    L = N
    for (k, s), (off, st) in zip(poolers, offs):
        L = (L - k) // s + 1
        sched.append((k, s, off, st, L))
    return sched


_MASK = _mask2d_np(_N, _POOLING_COUNTS)
_SCHED = _schedule(_N, _POOLING_COUNTS)


def _sc_body(xt_hbm, zeros_hbm, out_hbm, xv, obuf_a, obuf_b, sem_a, sem_b):
    c = lax.axis_index("c")
    s = lax.axis_index("s")
    b = s * _NC + c
    pltpu.make_async_copy(zeros_hbm, obuf_a, sem_a).start()
    pltpu.make_async_copy(zeros_hbm, obuf_b, sem_b).start()
    def unit(dc, carry):
        dt_in = dc // 8
        dl0_in = (dc % 8) * _LANES
        pltpu.sync_copy(xt_hbm.at[b, :, dt_in, :, pl.ds(dl0_in, _LANES)], xv)
        rows = [xv[n // 8, n % 8, :] for n in range(_N)]
        pltpu.make_async_copy(zeros_hbm, obuf_a, sem_a).wait()
        pltpu.make_async_copy(zeros_hbm, obuf_b, sem_b).wait()

        def put(v, i, j):
            if i < _N // 2:
                obuf_a[i, j // 8, j % 8, :] = v
            else:
                obuf_b[i - _N // 2, j // 8, j % 8, :] = v

        for i in range(_N):
            put(rows[i], i, i)
        cur = rows
        for k, sp, off, st, L in _SCHED:
            new = []
            for t in range(L):
                v = jnp.maximum(cur[sp * t], cur[sp * t + 1])
                if k == 3:
                    v = jnp.maximum(v, cur[sp * t + 2])
                new.append(v)
                put(v, st * t, st * t + off)
            cur = new
        dt = dc // 8
        dl0 = (dc % 8) * _LANES
        half = _N // 2
        dst_a = out_hbm.at[b, pl.ds(0, half), :, dt, :, pl.ds(dl0, _LANES)]
        dst_b = out_hbm.at[b, pl.ds(half, half), :, dt, :, pl.ds(dl0, _LANES)]
        pltpu.make_async_copy(obuf_a, dst_a, sem_a).start()
        pltpu.make_async_copy(obuf_b, dst_b, sem_b).start()
        return carry

    lax.fori_loop(0, _D // _LANES, unit, 0)
    pltpu.make_async_copy(zeros_hbm, obuf_a, sem_a).wait()
    pltpu.make_async_copy(zeros_hbm, obuf_b, sem_b).wait()


@functools.partial(jax.jit, static_argnames=())
def kernel(x):
    B, D, N = x.shape
    xt = x.reshape(B, D // 128, 128, N // 8, 8).transpose(0, 3, 1, 4, 2)
    zeros = jnp.zeros((N // 2, N // 8, 8, _LANES), dtype=x.dtype)
    mesh = plsc.VectorSubcoreMesh(
        core_axis_name="c", subcore_axis_name="s", num_cores=_NC, num_subcores=_NS
    )
    fn = pl.kernel(
        _sc_body,
        out_type=jax.ShapeDtypeStruct((B, N, N // 8, D // 128, 8, 128), x.dtype),
        mesh=mesh,
        scratch_types=[
            pltpu.VMEM((N // 8, 8, _LANES), x.dtype),
            pltpu.VMEM((N // 2, N // 8, 8, _LANES), x.dtype),
            pltpu.VMEM((N // 2, N // 8, 8, _LANES), x.dtype),
            pltpu.SemaphoreType.DMA,
            pltpu.SemaphoreType.DMA,
        ],
        compiler_params=pltpu.CompilerParams(
            needs_layout_passes=False, use_tc_tiling_on_sc=False
        ),
    )
    out6 = fn(xt, zeros)
    map2d = out6.transpose(0, 3, 5, 1, 2, 4).reshape(B, D, N, N)
    mask2d = jnp.broadcast_to(jnp.asarray(_MASK)[None, None, :, :], (B, 1, N, N))
    return (map2d, mask2d)


# speedup vs baseline: 1.5712x; 1.1019x over previous
import functools

import jax
import jax.numpy as jnp
import numpy as np
from jax import lax
from jax.experimental import pallas as pl
from jax.experimental.pallas import tpu as pltpu
from jax.experimental.pallas import tpu_sc as plsc

_POOLING_COUNTS = (15, 8, 8)
_N = 64
_B = 32
_D = 512
_NC = 2
_NS = 16
_LANES = 16


def _mask2d_np(N, pooling_counts):
    m = np.zeros((N, N), dtype=bool)
    m[np.arange(N), np.arange(N)] = True
    stride, offset = 1, 0
    for c in pooling_counts:
        for _ in range(c):
            offset += stride
            i = np.arange(0, N - offset, stride)
            m[i, i + offset] = True
        stride *= 2
    return m


def _schedule(N, pooling_counts):
    poolers = [(2, 1) for _ in range(pooling_counts[0])]
    for c in pooling_counts[1:]:
        poolers.append((3, 2))
        poolers.extend([(2, 1) for _ in range(c - 1)])
    offs = []
    stride, offset = 1, 0
    for c in pooling_counts:
        for _ in range(c):
            offset += stride
            offs.append((offset, stride))
        stride *= 2
    sched = []
    L = N
    for (k, s), (off, st) in zip(poolers, offs):
        L = (L - k) // s + 1
        sched.append((k, s, off, st, L))
    return sched


_MASK = _mask2d_np(_N, _POOLING_COUNTS)
_SCHED = _schedule(_N, _POOLING_COUNTS)


def _sc_body(xt_hbm, zeros_hbm, out_hbm, xv, obuf_a, obuf_b, sem_a, sem_b, sem_x):
    c = lax.axis_index("c")
    s = lax.axis_index("s")
    b = s * _NC + c
    pltpu.make_async_copy(zeros_hbm, obuf_a, sem_a).start()
    pltpu.make_async_copy(zeros_hbm, obuf_b, sem_b).start()

    def xsrc(dc):
        return xt_hbm.at[b, :, dc // 8, :, pl.ds((dc % 8) * _LANES, _LANES)]

    pltpu.make_async_copy(xsrc(0), xv, sem_x).start()

    def unit(dc, carry):
        pltpu.make_async_copy(xsrc(dc), xv, sem_x).wait()
        rows = [xv[n // 8, n % 8, :] for n in range(_N)]
        pltpu.make_async_copy(xsrc(jnp.minimum(dc + 1, 31)), xv, sem_x).start()
        pltpu.make_async_copy(zeros_hbm, obuf_a, sem_a).wait()
        pltpu.make_async_copy(zeros_hbm, obuf_b, sem_b).wait()

        def put(v, i, j):
            if i < _N // 2:
                obuf_a[i, j // 8, j % 8, :] = v
            else:
                obuf_b[i - _N // 2, j // 8, j % 8, :] = v

        for i in range(_N):
            put(rows[i], i, i)
        cur = rows
        for k, sp, off, st, L in _SCHED:
            new = []
            for t in range(L):
                v = jnp.maximum(cur[sp * t], cur[sp * t + 1])
                if k == 3:
                    v = jnp.maximum(v, cur[sp * t + 2])
                new.append(v)
                put(v, st * t, st * t + off)
            cur = new
        dt = dc // 8
        dl0 = (dc % 8) * _LANES
        half = _N // 2
        dst_a = out_hbm.at[b, pl.ds(0, half), :, dt, :, pl.ds(dl0, _LANES)]
        dst_b = out_hbm.at[b, pl.ds(half, half), :, dt, :, pl.ds(dl0, _LANES)]
        pltpu.make_async_copy(obuf_a, dst_a, sem_a).start()
        pltpu.make_async_copy(obuf_b, dst_b, sem_b).start()
        return carry

    lax.fori_loop(0, _D // _LANES, unit, 0)
    pltpu.make_async_copy(zeros_hbm, obuf_a, sem_a).wait()
    pltpu.make_async_copy(zeros_hbm, obuf_b, sem_b).wait()
    pltpu.make_async_copy(xsrc(31), xv, sem_x).wait()


@functools.partial(jax.jit, static_argnames=())
def kernel(x):
    B, D, N = x.shape
    xt = x.reshape(B, D // 128, 128, N // 8, 8).transpose(0, 3, 1, 4, 2)
    zeros = jnp.zeros((N // 2, N // 8, 8, _LANES), dtype=x.dtype)
    mesh = plsc.VectorSubcoreMesh(
        core_axis_name="c", subcore_axis_name="s", num_cores=_NC, num_subcores=_NS
    )
    fn = pl.kernel(
        _sc_body,
        out_type=jax.ShapeDtypeStruct((B, N, N // 8, D // 128, 8, 128), x.dtype),
        mesh=mesh,
        scratch_types=[
            pltpu.VMEM((N // 8, 8, _LANES), x.dtype),
            pltpu.VMEM((N // 2, N // 8, 8, _LANES), x.dtype),
            pltpu.VMEM((N // 2, N // 8, 8, _LANES), x.dtype),
            pltpu.SemaphoreType.DMA,
            pltpu.SemaphoreType.DMA,
            pltpu.SemaphoreType.DMA,
        ],
        compiler_params=pltpu.CompilerParams(
            needs_layout_passes=False, use_tc_tiling_on_sc=False
        ),
    )
    out6 = fn(xt, zeros)
    map2d = out6.transpose(0, 3, 5, 1, 2, 4).reshape(B, D, N, N)
    mask2d = jnp.broadcast_to(jnp.asarray(_MASK)[None, None, :, :], (B, 1, N, N))
    return (map2d, mask2d)
